# baseline (speedup 1.0000x reference)
import functools

import jax
import jax.numpy as jnp
from jax import lax
from jax.experimental import pallas as pl
from jax.experimental.pallas import tpu as pltpu
from jax.experimental.pallas import tpu_sc as plsc

_NUM_CLASSES = 1000000
_D = 64
_B = 16384
_LAMBDA_C = 0.5

_NC = 2
_NS = 16
_NW = _NC * _NS
_BPW = _B // _NW
_CHUNK = 128
_NCHUNK = _BPW // _CHUNK
_LANES = 16
_SCALE = _LAMBDA_C / (2.0 * _B)


def _body(idx_hbm, feat_hbm, centers_hbm, out_hbm,
          idx_v, rows_v, feat_v, res_v, gsem, fsem):
    wid = lax.axis_index("s") * _NC + lax.axis_index("c")
    base = wid * _BPW

    pltpu.sync_copy(idx_hbm.at[pl.ds(wid * _NCHUNK, _NCHUNK)], idx_v)

    fcopy = pltpu.async_copy(feat_hbm.at[pl.ds(base, _BPW)], feat_v, fsem)
    for j in range(_NCHUNK):
        pltpu.async_copy(
            centers_hbm.at[idx_v.at[j]],
            rows_v.at[pl.ds(j * _CHUNK, _CHUNK)],
            gsem,
        )
    fcopy.wait()
    for j in range(_NCHUNK):
        pltpu.make_async_copy(
            centers_hbm.at[idx_v.at[j]],
            rows_v.at[pl.ds(j * _CHUNK, _CHUNK)],
            gsem,
        ).wait()

    def row_step(r, acc):
        for c in range(_D // _LANES):
            f = feat_v[r, pl.ds(c * _LANES, _LANES)]
            g = rows_v[r, pl.ds(c * _LANES, _LANES)]
            d = f - g
            acc = acc + d * d
        return acc

    acc = lax.fori_loop(0, _BPW, row_step, jnp.zeros((_LANES,), jnp.float32))
    res_v[...] = acc * _SCALE
    pltpu.sync_copy(res_v, out_hbm.at[wid])


@jax.jit
def kernel(features, targets, centers):
    idx2d = targets.astype(jnp.int32).reshape(_B // _CHUNK, _CHUNK)
    run = functools.partial(
        pl.kernel,
        out_type=jax.ShapeDtypeStruct((_NW, _LANES), jnp.float32),
        mesh=plsc.VectorSubcoreMesh(core_axis_name="c", subcore_axis_name="s"),
        scratch_types=[
            pltpu.VMEM((_NCHUNK, _CHUNK), jnp.int32),
            pltpu.VMEM((_BPW, _D), jnp.float32),
            pltpu.VMEM((_BPW, _D), jnp.float32),
            pltpu.VMEM((_LANES,), jnp.float32),
            pltpu.SemaphoreType.DMA,
            pltpu.SemaphoreType.DMA,
        ],
        compiler_params=pltpu.CompilerParams(use_tc_tiling_on_sc=False),
    )(_body)
    partials = run(idx2d, features, centers)
    return jnp.sum(partials)


# speedup vs baseline: 1.6542x; 1.6542x over previous
import functools

import jax
import jax.numpy as jnp
from jax import lax
from jax.experimental import pallas as pl
from jax.experimental.pallas import tpu as pltpu
from jax.experimental.pallas import tpu_sc as plsc

_NUM_CLASSES = 1000000
_D = 64
_B = 16384
_LAMBDA_C = 0.5

_NC = 2
_NS = 16
_NW = _NC * _NS
_BPW = _B // _NW
_W = 16
_NWAVE = _BPW // _W
_LANES = 16
_SCALE = _LAMBDA_C / (2.0 * _B)


def _body(idx_hbm, feat_hbm, centers_hbm, out_hbm,
          idx_v, rows_v, feat_v, res_v, gsem, fsem):
    wid = lax.axis_index("s") * _NC + lax.axis_index("c")
    base = wid * _BPW

    pltpu.sync_copy(idx_hbm.at[pl.ds(base, _BPW)], idx_v)
    fcopy = pltpu.async_copy(feat_hbm.at[pl.ds(base, _BPW)], feat_v, fsem)

    def fire(w, buf):
        tv = idx_v[pl.ds(w * _W, _W)]
        for k in range(_W):
            t = tv[k]
            pltpu.async_copy(
                centers_hbm.at[pl.ds(t, 1)],
                rows_v.at[buf, pl.ds(k, 1)],
                gsem.at[buf],
            )

    def drain(buf):
        pltpu.make_async_copy(
            centers_hbm.at[pl.ds(0, _W)],
            rows_v.at[buf],
            gsem.at[buf],
        ).wait()

    def compute(w, buf, acc):
        for r in range(_W):
            for c in range(_D // _LANES):
                f = feat_v[w * _W + r, pl.ds(c * _LANES, _LANES)]
                g = rows_v[buf, r, pl.ds(c * _LANES, _LANES)]
                d = f - g
                acc = acc + d * d
        return acc

    fire(0, 0)
    fcopy.wait()

    def wave_step(w, acc):
        buf = lax.rem(w, 2)
        fire(w, buf)
        pbuf = lax.rem(w + 1, 2)
        drain(pbuf)
        return compute(w - 1, pbuf, acc)

    acc = lax.fori_loop(1, _NWAVE, wave_step,
                        jnp.zeros((_LANES,), jnp.float32))
    last = _NWAVE - 1
    lbuf = last % 2
    drain(lbuf)
    acc = compute(last, lbuf, acc)

    res_v[...] = acc * _SCALE
    pltpu.sync_copy(res_v, out_hbm.at[wid])


@jax.jit
def kernel(features, targets, centers):
    idx = targets.astype(jnp.int32)
    run = functools.partial(
        pl.kernel,
        out_type=jax.ShapeDtypeStruct((_NW, _LANES), jnp.float32),
        mesh=plsc.VectorSubcoreMesh(core_axis_name="c", subcore_axis_name="s"),
        scratch_types=[
            pltpu.VMEM((_BPW,), jnp.int32),
            pltpu.VMEM((2, _W, _D), jnp.float32),
            pltpu.VMEM((_BPW, _D), jnp.float32),
            pltpu.VMEM((_LANES,), jnp.float32),
            pltpu.SemaphoreType.DMA((2,)),
            pltpu.SemaphoreType.DMA,
        ],
    )(_body)
    partials = run(idx, features, centers)
    return jnp.sum(partials)
